# Initial kernel scaffold; baseline (speedup 1.0000x reference)
import functools

import jax
import jax.numpy as jnp
from jax import lax
from jax.experimental import pallas as pl
from jax.experimental.pallas import tpu as pltpu
from jax.experimental.pallas import tpu_sc as plsc

_B = 16384
_V = 1000000
_E = 64
_P = 10
_N = 50
_NCTX = _P + _N
_SW = 64

_info = plsc.get_sparse_core_info()
_NC = _info.num_cores
_NS = _info.num_subcores
_L = _info.num_lanes
_NW = _NC * _NS
_BPW = _B // _NW
_C = 16
_CHUNKS = _BPW // _C

_mesh = plsc.VectorSubcoreMesh(core_axis_name="c", subcore_axis_name="s")


@functools.partial(
    pl.kernel,
    mesh=_mesh,
    out_type=jax.ShapeDtypeStruct((_B * _SW,), jnp.float32),
    scratch_types=[
        pltpu.VMEM((_C,), jnp.int32),
        pltpu.VMEM((_C, _E), jnp.float32),
        pltpu.VMEM((_C * _NCTX,), jnp.int32),
        pltpu.VMEM((_C * _NCTX, _E), jnp.float32),
        pltpu.VMEM((_C * _SW,), jnp.float32),
        pltpu.SemaphoreType.DMA,
    ],
)
def _sc_scores(center_hbm, ctx_hbm, in_embed_hbm, out_embed_hbm,
               scores_hbm, cidx_v, crows_v, idx_v, rows_v, scores_v, sem):
    wid = lax.axis_index("s") * _NC + lax.axis_index("c")
    base = wid * _BPW
    lane = lax.iota(jnp.int32, _L)

    def chunk_body(t, carry):
        b0 = base + t * _C
        pltpu.sync_copy(center_hbm.at[pl.ds(b0, _C)], cidx_v)
        pltpu.sync_copy(ctx_hbm.at[pl.ds(b0 * _NCTX, _C * _NCTX)], idx_v)
        pltpu.async_copy(in_embed_hbm.at[cidx_v], crows_v, sem).wait()
        pltpu.async_copy(out_embed_hbm.at[idx_v], rows_v, sem).wait()

        def elem_body(i, carry2):
            c0 = crows_v[i, pl.ds(0, 16)]
            c1 = crows_v[i, pl.ds(16, 16)]
            c2 = crows_v[i, pl.ds(32, 16)]
            c3 = crows_v[i, pl.ds(48, 16)]
            r0 = i * _NCTX
            s0 = i * _SW
            for g in range(4):
                nj = 16 if g < 3 else _NCTX - 48
                acc = jnp.zeros((_L,), jnp.float32)
                for l in range(nj):
                    j = g * 16 + l
                    u0 = rows_v[r0 + j, pl.ds(0, 16)]
                    u1 = rows_v[r0 + j, pl.ds(16, 16)]
                    u2 = rows_v[r0 + j, pl.ds(32, 16)]
                    u3 = rows_v[r0 + j, pl.ds(48, 16)]
                    s = jnp.sum(c0 * u0 + c1 * u1 + c2 * u2 + c3 * u3)
                    acc = jnp.where(lane == l, s, acc)
                scores_v[pl.ds(s0 + g * 16, 16)] = acc
            return carry2

        lax.fori_loop(0, _C, elem_body, 0)
        pltpu.sync_copy(scores_v, scores_hbm.at[pl.ds(b0 * _SW, _C * _SW)])
        return carry

    lax.fori_loop(0, _CHUNKS, chunk_body, 0)


def _tc_loss_body(scores_ref, out_ref):
    s = scores_ref[...]
    col = lax.broadcasted_iota(jnp.int32, s.shape, 1)
    x = jnp.where(col < _P, s, -s)
    ls = jnp.minimum(x, 0.0) - jnp.log1p(jnp.exp(-jnp.abs(x)))
    val = jnp.where(col < _NCTX, ls, 0.0)
    out_ref[...] = -jnp.sum(val, axis=1, keepdims=True)


_tc_loss = pl.pallas_call(
    _tc_loss_body,
    out_shape=jax.ShapeDtypeStruct((_B, 1), jnp.float32),
)


def kernel(center_word, target_word, negative_word, in_embed, out_embed):
    center = center_word.astype(jnp.int32)
    ctx = jnp.concatenate(
        [target_word.astype(jnp.int32), negative_word.astype(jnp.int32)], axis=1)
    ctx_flat = ctx.reshape(-1)
    scores_flat = _sc_scores(center, ctx_flat, in_embed, out_embed)
    scores = scores_flat.reshape(_B, _SW)
    loss = _tc_loss(scores)
    return loss.reshape(_B)


# speedup vs baseline: 2.3278x; 2.3278x over previous
import functools

import jax
import jax.numpy as jnp
from jax import lax
from jax.experimental import pallas as pl
from jax.experimental.pallas import tpu as pltpu
from jax.experimental.pallas import tpu_sc as plsc

_B = 16384
_V = 1000000
_E = 64
_P = 10
_N = 50
_NCTX = _P + _N
_SW = 64

_NC = 2
_NS = 16
_L = 16
_NW = _NC * _NS
_BPW = _B // _NW
_C = 16
_CHUNKS = _BPW // _C

def _sc_scores_body(center_hbm, ctx_hbm, in_embed_hbm, out_embed_hbm,
                    scores_hbm, cidx_v, crows_v, idx_v, rows_v, scores_v, sem):
    wid = lax.axis_index("s") * _NC + lax.axis_index("c")
    base = wid * _BPW
    lane = lax.iota(jnp.int32, _L)

    def chunk_body(t, carry):
        b0 = base + t * _C
        pltpu.sync_copy(center_hbm.at[pl.ds(b0, _C)], cidx_v)
        pltpu.sync_copy(ctx_hbm.at[pl.ds(b0 * _NCTX, _C * _NCTX)], idx_v)
        pltpu.async_copy(in_embed_hbm.at[cidx_v], crows_v, sem).wait()
        pltpu.async_copy(out_embed_hbm.at[idx_v], rows_v, sem).wait()

        def elem_body(i, carry2):
            r0 = i * _NCTX
            s0 = i * _SW
            cvecs = [crows_v[i, pl.ds(16 * q, 16)] for q in range(4)]
            for g in range(4):
                jvec = r0 + g * 16 + lane
                if g == 3:
                    jvec = jnp.minimum(jvec, r0 + _NCTX - 1)
                acc = jnp.zeros((_L,), jnp.float32)
                for e in range(_E):
                    evec = jnp.full((_L,), e, jnp.int32)
                    u = plsc.load_gather(rows_v, [jvec, evec])
                    acc = acc + u * cvecs[e // 16][e % 16]
                scores_v[pl.ds(s0 + g * 16, 16)] = acc
            return carry2

        lax.fori_loop(0, _C, elem_body, 0)
        pltpu.sync_copy(scores_v, scores_hbm.at[pl.ds(b0 * _SW, _C * _SW)])
        return carry

    lax.fori_loop(0, _CHUNKS, chunk_body, 0)


@functools.cache
def _sc_scores_fn():
    mesh = plsc.VectorSubcoreMesh(
        core_axis_name="c", subcore_axis_name="s", num_cores=_NC)
    return pl.kernel(
        _sc_scores_body,
        mesh=mesh,
        compiler_params=pltpu.CompilerParams(
            needs_layout_passes=False, use_tc_tiling_on_sc=False),
        out_type=jax.ShapeDtypeStruct((_B * _SW,), jnp.float32),
        scratch_types=[
            pltpu.VMEM((_C,), jnp.int32),
            pltpu.VMEM((_C, _E), jnp.float32),
            pltpu.VMEM((_C * _NCTX,), jnp.int32),
            pltpu.VMEM((_C * _NCTX, _E), jnp.float32),
            pltpu.VMEM((_C * _SW,), jnp.float32),
            pltpu.SemaphoreType.DMA,
        ],
    )


def _tc_loss_body(scores_ref, out_ref):
    s = scores_ref[...]
    col = lax.broadcasted_iota(jnp.int32, s.shape, 1)
    x = jnp.where(col < _P, s, -s)
    ls = jnp.minimum(x, 0.0) - jnp.log1p(jnp.exp(-jnp.abs(x)))
    val = jnp.where(col < _NCTX, ls, 0.0)
    out_ref[...] = -jnp.sum(val, axis=1, keepdims=True)


_tc_loss = pl.pallas_call(
    _tc_loss_body,
    out_shape=jax.ShapeDtypeStruct((_B, 1), jnp.float32),
)


def kernel(center_word, target_word, negative_word, in_embed, out_embed):
    center = center_word.astype(jnp.int32)
    ctx = jnp.concatenate(
        [target_word.astype(jnp.int32), negative_word.astype(jnp.int32)], axis=1)
    ctx_flat = ctx.reshape(-1)
    scores_flat = _sc_scores_fn()(center, ctx_flat, in_embed, out_embed)
    scores = scores_flat.reshape(_B, _SW)
    loss = _tc_loss(scores)
    return loss.reshape(_B)


# speedup vs baseline: 2.5375x; 1.0901x over previous
import functools

import jax
import jax.numpy as jnp
from jax import lax
from jax.experimental import pallas as pl
from jax.experimental.pallas import tpu as pltpu
from jax.experimental.pallas import tpu_sc as plsc

_B = 16384
_V = 1000000
_E = 64
_P = 10
_N = 50
_NCTX = _P + _N
_SW = 64

_NC = 2
_NS = 16
_L = 16
_NW = _NC * _NS
_BPW = _B // _NW
_C = 16
_CHUNKS = _BPW // _C

def _sc_scores_body(center_hbm, ctx_hbm, in_embed_hbm, out_embed_hbm,
                    scores_hbm,
                    cidx0, cidx1, crows0, crows1, idx0, idx1, rows0, rows1,
                    scores_v, semi0, semi1, semg0, semg1):
    wid = lax.axis_index("s") * _NC + lax.axis_index("c")
    base = wid * _BPW
    lane = lax.iota(jnp.int32, _L)
    cidx = (cidx0, cidx1)
    crows = (crows0, crows1)
    idx = (idx0, idx1)
    rows = (rows0, rows1)
    semi = (semi0, semi1)
    semg = (semg0, semg1)

    def fire_idx(c, b):
        b0 = base + c * _C
        pltpu.async_copy(
            ctx_hbm.at[pl.ds(b0 * _NCTX, _C * _NCTX)], idx[b], semi[b])
        pltpu.async_copy(center_hbm.at[pl.ds(b0, _C)], cidx[b], semi[b])

    def wait_idx(c, b):
        b0 = base + c * _C
        pltpu.make_async_copy(
            ctx_hbm.at[pl.ds(b0 * _NCTX, _C * _NCTX)], idx[b], semi[b]).wait()
        pltpu.make_async_copy(
            center_hbm.at[pl.ds(b0, _C)], cidx[b], semi[b]).wait()

    def fire_gather(b):
        pltpu.async_copy(out_embed_hbm.at[idx[b]], rows[b], semg[b])
        pltpu.async_copy(in_embed_hbm.at[cidx[b]], crows[b], semg[b])

    def wait_gather(b):
        pltpu.make_async_copy(out_embed_hbm.at[idx[b]], rows[b], semg[b]).wait()
        pltpu.make_async_copy(in_embed_hbm.at[cidx[b]], crows[b], semg[b]).wait()

    def compute(t, b):
        rv, cv = rows[b], crows[b]

        def elem_body(i, carry2):
            r0 = i * _NCTX
            s0 = i * _SW
            cvecs = [cv[i, pl.ds(16 * q, 16)] for q in range(4)]
            for g in range(4):
                jvec = r0 + g * 16 + lane
                if g == 3:
                    jvec = jnp.minimum(jvec, r0 + _NCTX - 1)
                accs = [jnp.zeros((_L,), jnp.float32) for _ in range(4)]
                for e in range(_E):
                    evec = jnp.full((_L,), e, jnp.int32)
                    u = plsc.load_gather(rv, [jvec, evec])
                    accs[e % 4] = accs[e % 4] + u * cvecs[e // 16][e % 16]
                scores_v[pl.ds(s0 + g * 16, 16)] = (
                    (accs[0] + accs[1]) + (accs[2] + accs[3]))
            return carry2

        lax.fori_loop(0, _C, elem_body, 0)
        b0 = base + t * _C
        pltpu.sync_copy(scores_v, scores_hbm.at[pl.ds(b0 * _SW, _C * _SW)])

    fire_idx(0, 0)
    wait_idx(0, 0)
    fire_gather(0)
    fire_idx(1, 1)

    def pair_body(k, carry):
        for b in (0, 1):
            t = 2 * k + b
            nb = 1 - b

            @pl.when(t + 1 < _CHUNKS)
            def _():
                wait_idx(t + 1, nb)
                fire_gather(nb)

            wait_gather(b)

            @pl.when(t + 2 < _CHUNKS)
            def _():
                fire_idx(t + 2, b)

            compute(t, b)
        return carry

    lax.fori_loop(0, _CHUNKS // 2, pair_body, 0)


@functools.cache
def _sc_scores_fn():
    mesh = plsc.VectorSubcoreMesh(
        core_axis_name="c", subcore_axis_name="s", num_cores=_NC)
    return pl.kernel(
        _sc_scores_body,
        mesh=mesh,
        compiler_params=pltpu.CompilerParams(
            needs_layout_passes=False, use_tc_tiling_on_sc=False),
        out_type=jax.ShapeDtypeStruct((_B * _SW,), jnp.float32),
        scratch_types=[
            pltpu.VMEM((_C,), jnp.int32),
            pltpu.VMEM((_C,), jnp.int32),
            pltpu.VMEM((_C, _E), jnp.float32),
            pltpu.VMEM((_C, _E), jnp.float32),
            pltpu.VMEM((_C * _NCTX,), jnp.int32),
            pltpu.VMEM((_C * _NCTX,), jnp.int32),
            pltpu.VMEM((_C * _NCTX, _E), jnp.float32),
            pltpu.VMEM((_C * _NCTX, _E), jnp.float32),
            pltpu.VMEM((_C * _SW,), jnp.float32),
            pltpu.SemaphoreType.DMA,
            pltpu.SemaphoreType.DMA,
            pltpu.SemaphoreType.DMA,
            pltpu.SemaphoreType.DMA,
        ],
    )


def _tc_loss_body(scores_ref, out_ref):
    s = scores_ref[...]
    col = lax.broadcasted_iota(jnp.int32, s.shape, 1)
    x = jnp.where(col < _P, s, -s)
    ls = jnp.minimum(x, 0.0) - jnp.log1p(jnp.exp(-jnp.abs(x)))
    val = jnp.where(col < _NCTX, ls, 0.0)
    out_ref[...] = -jnp.sum(val, axis=1, keepdims=True)


_tc_loss = pl.pallas_call(
    _tc_loss_body,
    out_shape=jax.ShapeDtypeStruct((_B, 1), jnp.float32),
)


def kernel(center_word, target_word, negative_word, in_embed, out_embed):
    center = center_word.astype(jnp.int32)
    ctx = jnp.concatenate(
        [target_word.astype(jnp.int32), negative_word.astype(jnp.int32)], axis=1)
    ctx_flat = ctx.reshape(-1)
    scores_flat = _sc_scores_fn()(center, ctx_flat, in_embed, out_embed)
    scores = scores_flat.reshape(_B, _SW)
    loss = _tc_loss(scores)
    return loss.reshape(_B)


# speedup vs baseline: 3.9973x; 1.5753x over previous
import functools

import jax
import jax.numpy as jnp
from jax import lax
from jax.experimental import pallas as pl
from jax.experimental.pallas import tpu as pltpu
from jax.experimental.pallas import tpu_sc as plsc

_B = 16384
_V = 1000000
_E = 64
_P = 10
_N = 50
_NCTX = _P + _N
_SW = 64

_NC = 2
_NS = 16
_L = 16
_NW = _NC * _NS
_BPW = _B // _NW
_C = 16
_CHUNKS = _BPW // _C

def _sc_scores_body(center_hbm, ctx_hbm, in_embed_hbm, out_embed_hbm,
                    scores_hbm,
                    cidx0, cidx1, crows0, crows1, idx0, idx1, rows0, rows1,
                    scores_v, semi0, semi1, semg0, semg1):
    wid = lax.axis_index("s") * _NC + lax.axis_index("c")
    base = wid * _BPW
    lane = lax.iota(jnp.int32, _L)
    cidx = (cidx0, cidx1)
    crows = (crows0, crows1)
    idx = (idx0, idx1)
    rows = (rows0, rows1)
    semi = (semi0, semi1)
    semg = (semg0, semg1)

    def fire_idx(c, b):
        b0 = base + c * _C
        pltpu.async_copy(
            ctx_hbm.at[pl.ds(b0 * _NCTX, _C * _NCTX)], idx[b], semi[b])
        pltpu.async_copy(center_hbm.at[pl.ds(b0, _C)], cidx[b], semi[b])

    def wait_idx(c, b):
        b0 = base + c * _C
        pltpu.make_async_copy(
            ctx_hbm.at[pl.ds(b0 * _NCTX, _C * _NCTX)], idx[b], semi[b]).wait()
        pltpu.make_async_copy(
            center_hbm.at[pl.ds(b0, _C)], cidx[b], semi[b]).wait()

    def fire_gather(b):
        pltpu.async_copy(out_embed_hbm.at[idx[b]], rows[b], semg[b])
        pltpu.async_copy(in_embed_hbm.at[cidx[b]], crows[b], semg[b])

    def wait_gather(b):
        pltpu.make_async_copy(out_embed_hbm.at[idx[b]], rows[b], semg[b]).wait()
        pltpu.make_async_copy(in_embed_hbm.at[cidx[b]], crows[b], semg[b]).wait()

    def compute(t, b):
        rv, cv = rows[b], crows[b]

        def elem_body(i, carry2):
            r0 = i * _NCTX
            s0 = i * _SW
            ivec = jnp.zeros((_L,), jnp.int32) + i
            jvecs = []
            for g in range(4):
                jv = r0 + g * 16 + lane
                if g == 3:
                    jv = jnp.minimum(jv, r0 + _NCTX - 1)
                jvecs.append(jv)
            accs = [[jnp.zeros((_L,), jnp.float32) for _ in range(2)]
                    for _ in range(4)]
            for e0 in range(_E):
                evec = (lane + e0) & (_E - 1)
                c_l = plsc.load_gather(cv, [ivec, evec])
                for g in range(4):
                    u = plsc.load_gather(rv, [jvecs[g], evec])
                    accs[g][e0 % 2] = accs[g][e0 % 2] + u * c_l
            for g in range(4):
                scores_v[pl.ds(s0 + g * 16, 16)] = accs[g][0] + accs[g][1]
            return carry2

        lax.fori_loop(0, _C, elem_body, 0)
        b0 = base + t * _C
        pltpu.sync_copy(scores_v, scores_hbm.at[pl.ds(b0 * _SW, _C * _SW)])

    fire_idx(0, 0)
    wait_idx(0, 0)
    fire_gather(0)
    fire_idx(1, 1)

    def pair_body(k, carry):
        for b in (0, 1):
            t = 2 * k + b
            nb = 1 - b

            @pl.when(t + 1 < _CHUNKS)
            def _():
                wait_idx(t + 1, nb)
                fire_gather(nb)

            wait_gather(b)

            @pl.when(t + 2 < _CHUNKS)
            def _():
                fire_idx(t + 2, b)

            compute(t, b)
        return carry

    lax.fori_loop(0, _CHUNKS // 2, pair_body, 0)


@functools.cache
def _sc_scores_fn():
    mesh = plsc.VectorSubcoreMesh(
        core_axis_name="c", subcore_axis_name="s", num_cores=_NC)
    return pl.kernel(
        _sc_scores_body,
        mesh=mesh,
        compiler_params=pltpu.CompilerParams(
            needs_layout_passes=False, use_tc_tiling_on_sc=False),
        out_type=jax.ShapeDtypeStruct((_B * _SW,), jnp.float32),
        scratch_types=[
            pltpu.VMEM((_C,), jnp.int32),
            pltpu.VMEM((_C,), jnp.int32),
            pltpu.VMEM((_C, _E), jnp.float32),
            pltpu.VMEM((_C, _E), jnp.float32),
            pltpu.VMEM((_C * _NCTX,), jnp.int32),
            pltpu.VMEM((_C * _NCTX,), jnp.int32),
            pltpu.VMEM((_C * _NCTX, _E), jnp.float32),
            pltpu.VMEM((_C * _NCTX, _E), jnp.float32),
            pltpu.VMEM((_C * _SW,), jnp.float32),
            pltpu.SemaphoreType.DMA,
            pltpu.SemaphoreType.DMA,
            pltpu.SemaphoreType.DMA,
            pltpu.SemaphoreType.DMA,
        ],
    )


def _tc_loss_body(scores_ref, out_ref):
    s = scores_ref[...]
    col = lax.broadcasted_iota(jnp.int32, s.shape, 1)
    x = jnp.where(col < _P, s, -s)
    ls = jnp.minimum(x, 0.0) - jnp.log1p(jnp.exp(-jnp.abs(x)))
    val = jnp.where(col < _NCTX, ls, 0.0)
    out_ref[...] = -jnp.sum(val, axis=1, keepdims=True)


_tc_loss = pl.pallas_call(
    _tc_loss_body,
    out_shape=jax.ShapeDtypeStruct((_B, 1), jnp.float32),
)


def kernel(center_word, target_word, negative_word, in_embed, out_embed):
    center = center_word.astype(jnp.int32)
    ctx = jnp.concatenate(
        [target_word.astype(jnp.int32), negative_word.astype(jnp.int32)], axis=1)
    ctx_flat = ctx.reshape(-1)
    scores_flat = _sc_scores_fn()(center, ctx_flat, in_embed, out_embed)
    scores = scores_flat.reshape(_B, _SW)
    loss = _tc_loss(scores)
    return loss.reshape(_B)


# speedup vs baseline: 3.9987x; 1.0004x over previous
import functools

import jax
import jax.numpy as jnp
from jax import lax
from jax.experimental import pallas as pl
from jax.experimental.pallas import tpu as pltpu
from jax.experimental.pallas import tpu_sc as plsc

_B = 16384
_V = 1000000
_E = 64
_P = 10
_N = 50
_NCTX = _P + _N
_SW = 64

_NC = 2
_NS = 16
_L = 16
_NW = _NC * _NS
_BPW = _B // _NW
_C = 16
_CHUNKS = _BPW // _C

def _sc_scores_body(center_hbm, ctx_hbm, in_embed_hbm, out_embed_hbm,
                    scores_hbm,
                    cidx0, cidx1, crows0, crows1, idx0, idx1, rows0, rows1,
                    scores_v, semi0, semi1, semg0, semg1):
    wid = lax.axis_index("s") * _NC + lax.axis_index("c")
    base = wid * _BPW
    lane = lax.iota(jnp.int32, _L)
    cidx = (cidx0, cidx1)
    crows = (crows0, crows1)
    idx = (idx0, idx1)
    rows = (rows0, rows1)
    semi = (semi0, semi1)
    semg = (semg0, semg1)

    def fire_idx(c, b):
        b0 = base + c * _C
        pltpu.async_copy(
            ctx_hbm.at[pl.ds(b0 * _NCTX, _C * _NCTX)], idx[b], semi[b])
        pltpu.async_copy(center_hbm.at[pl.ds(b0, _C)], cidx[b], semi[b])

    def wait_idx(c, b):
        b0 = base + c * _C
        pltpu.make_async_copy(
            ctx_hbm.at[pl.ds(b0 * _NCTX, _C * _NCTX)], idx[b], semi[b]).wait()
        pltpu.make_async_copy(
            center_hbm.at[pl.ds(b0, _C)], cidx[b], semi[b]).wait()

    def fire_gather(b):
        pltpu.async_copy(out_embed_hbm.at[idx[b]], rows[b], semg[b])
        pltpu.async_copy(in_embed_hbm.at[cidx[b]], crows[b], semg[b])

    def wait_gather(b):
        pltpu.make_async_copy(out_embed_hbm.at[idx[b]], rows[b], semg[b]).wait()
        pltpu.make_async_copy(in_embed_hbm.at[cidx[b]], crows[b], semg[b]).wait()

    def compute(t, b):
        rv, cv = rows[b], crows[b]

        def elem_body(i, carry2):
            r0 = i * _NCTX
            s0 = i * _SW
            ivec = jnp.zeros((_L,), jnp.int32) + i
            jvecs = []
            for g in range(4):
                jv = r0 + g * 16 + lane
                if g == 3:
                    jv = jnp.minimum(jv, r0 + _NCTX - 1)
                jvecs.append(jv)
            accs = [[jnp.zeros((_L,), jnp.float32) for _ in range(2)]
                    for _ in range(4)]
            for e0 in range(_E):
                evec = (lane + e0) & (_E - 1)
                c_l = plsc.load_gather(cv, [ivec, evec])
                for g in range(4):
                    u = plsc.load_gather(rv, [jvecs[g], evec])
                    accs[g][e0 % 2] = accs[g][e0 % 2] + u * c_l
            for g in range(4):
                scores_v[pl.ds(s0 + g * 16, 16)] = accs[g][0] + accs[g][1]
            return carry2

        lax.fori_loop(0, _C, elem_body, 0)
        b0 = base + t * _C
        pltpu.sync_copy(scores_v, scores_hbm.at[pl.ds(b0 * _SW, _C * _SW)])

    fire_idx(0, 0)
    wait_idx(0, 0)
    fire_gather(0)
    fire_idx(1, 1)

    def pair_body(k, carry):
        for b in (0, 1):
            t = 2 * k + b
            nb = 1 - b

            @pl.when(t + 1 < _CHUNKS)
            def _():
                wait_idx(t + 1, nb)
                fire_gather(nb)

            wait_gather(b)

            @pl.when(t + 2 < _CHUNKS)
            def _():
                fire_idx(t + 2, b)

            compute(t, b)
        return carry

    lax.fori_loop(0, _CHUNKS // 2, pair_body, 0)


@functools.cache
def _sc_scores_fn():
    mesh = plsc.VectorSubcoreMesh(
        core_axis_name="c", subcore_axis_name="s", num_cores=_NC)
    return pl.kernel(
        _sc_scores_body,
        mesh=mesh,
        compiler_params=pltpu.CompilerParams(
            needs_layout_passes=False, use_tc_tiling_on_sc=False),
        out_type=jax.ShapeDtypeStruct((_B * _SW,), jnp.float32),
        scratch_types=[
            pltpu.VMEM((_C,), jnp.int32),
            pltpu.VMEM((_C,), jnp.int32),
            pltpu.VMEM((_C, _E), jnp.float32),
            pltpu.VMEM((_C, _E), jnp.float32),
            pltpu.VMEM((_C * _NCTX,), jnp.int32),
            pltpu.VMEM((_C * _NCTX,), jnp.int32),
            pltpu.VMEM((_C * _NCTX, _E), jnp.float32),
            pltpu.VMEM((_C * _NCTX, _E), jnp.float32),
            pltpu.VMEM((_C * _SW,), jnp.float32),
            pltpu.SemaphoreType.DMA,
            pltpu.SemaphoreType.DMA,
            pltpu.SemaphoreType.DMA,
            pltpu.SemaphoreType.DMA,
        ],
    )


def _tc_loss_body(scores_ref, out_ref):
    s = scores_ref[...]
    col = lax.broadcasted_iota(jnp.int32, s.shape, 1)
    x = jnp.where(col < _P, s, -s)
    ls = jnp.minimum(x, 0.0) - jnp.log1p(jnp.exp(-jnp.abs(x)))
    val = jnp.where(col < _NCTX, ls, 0.0)
    out_ref[...] = -jnp.sum(val, axis=1, keepdims=True)


_tc_loss = pl.pallas_call(
    _tc_loss_body,
    out_shape=jax.ShapeDtypeStruct((_B, 1), jnp.float32),
)


def kernel(center_word, target_word, negative_word, in_embed, out_embed):
    center = center_word.astype(jnp.int32)
    ctx = jnp.concatenate(
        [target_word.astype(jnp.int32), negative_word.astype(jnp.int32)], axis=1)
    ctx_flat = ctx.reshape(-1)
    in_flat = lax.optimization_barrier(jnp.ravel(in_embed)).reshape(_V, _E)
    out_flat = lax.optimization_barrier(jnp.ravel(out_embed)).reshape(_V, _E)
    scores_flat = _sc_scores_fn()(center, ctx_flat, in_flat, out_flat)
    scores = scores_flat.reshape(_B, _SW)
    loss = _tc_loss(scores)
    return loss.reshape(_B)


# speedup vs baseline: 4.6365x; 1.1595x over previous
import functools

import jax
import jax.numpy as jnp
from jax import lax
from jax.experimental import pallas as pl
from jax.experimental.pallas import tpu as pltpu
from jax.experimental.pallas import tpu_sc as plsc

_B = 16384
_V = 1000000
_E = 64
_P = 10
_N = 50
_NCTX = _P + _N
_SW = 64

_NC = 2
_NS = 16
_L = 16
_NW = _NC * _NS
_BPW = _B // _NW
_C = 16
_CHUNKS = _BPW // _C

def _sc_scores_body(center_hbm, ctx_hbm, in_embed_hbm, out_embed_hbm,
                    scores_hbm,
                    cidx0, cidx1, crows0, crows1, idx0, idx1, rows0, rows1,
                    scores_v, semi0, semi1, semg0, semg1):
    wid = lax.axis_index("s") * _NC + lax.axis_index("c")
    base = wid * _BPW
    lane = lax.iota(jnp.int32, _L)
    cidx = (cidx0, cidx1)
    crows = (crows0, crows1)
    idx = (idx0, idx1)
    rows = (rows0, rows1)
    semi = (semi0, semi1)
    semg = (semg0, semg1)

    def fire_idx(c, b):
        b0 = base + c * _C
        pltpu.async_copy(
            ctx_hbm.at[pl.ds(b0 * _NCTX, _C * _NCTX)], idx[b], semi[b])
        pltpu.async_copy(center_hbm.at[pl.ds(b0, _C)], cidx[b], semi[b])

    def wait_idx(c, b):
        b0 = base + c * _C
        pltpu.make_async_copy(
            ctx_hbm.at[pl.ds(b0 * _NCTX, _C * _NCTX)], idx[b], semi[b]).wait()
        pltpu.make_async_copy(
            center_hbm.at[pl.ds(b0, _C)], cidx[b], semi[b]).wait()

    def fire_gather(b):
        pltpu.async_copy(out_embed_hbm.at[idx[b]], rows[b], semg[b])
        pltpu.async_copy(in_embed_hbm.at[cidx[b]], crows[b], semg[b])

    def wait_gather(b):
        pltpu.make_async_copy(out_embed_hbm.at[idx[b]], rows[b], semg[b]).wait()
        pltpu.make_async_copy(in_embed_hbm.at[cidx[b]], crows[b], semg[b]).wait()

    def compute(t, b):
        rv, cv = rows[b], crows[b]

        def elem_body(i, carry2):
            r0 = i * _NCTX
            s0 = i * _SW
            ivec = jnp.zeros((_L,), jnp.int32) + i
            jvecs = []
            for g in range(4):
                jv = r0 + g * 16 + lane
                if g == 3:
                    jv = jnp.minimum(jv, r0 + _NCTX - 1)
                jvecs.append(jv)
            accs = [[jnp.zeros((_L,), jnp.float32) for _ in range(2)]
                    for _ in range(4)]
            for e0 in range(_E):
                evec = (lane + e0) & (_E - 1)
                c_l = plsc.load_gather(cv, [ivec, evec])
                for g in range(4):
                    u = plsc.load_gather(rv, [jvecs[g], evec])
                    accs[g][e0 % 2] = accs[g][e0 % 2] + u * c_l
            for g in range(4):
                scores_v[pl.ds(s0 + g * 16, 16)] = accs[g][0] + accs[g][1]
            return carry2

        lax.fori_loop(0, _C, elem_body, 0)
        b0 = base + t * _C
        pltpu.sync_copy(scores_v, scores_hbm.at[pl.ds(b0 * _SW, _C * _SW)])

    fire_idx(0, 0)
    wait_idx(0, 0)
    fire_gather(0)
    fire_idx(1, 1)

    def pair_body(k, carry):
        for b in (0, 1):
            t = 2 * k + b
            nb = 1 - b

            @pl.when(t + 1 < _CHUNKS)
            def _():
                wait_idx(t + 1, nb)
                fire_gather(nb)

            wait_gather(b)

            @pl.when(t + 2 < _CHUNKS)
            def _():
                fire_idx(t + 2, b)

            compute(t, b)
        return carry

    lax.fori_loop(0, _CHUNKS // 2, pair_body, 0)


@functools.cache
def _sc_scores_fn():
    mesh = plsc.VectorSubcoreMesh(
        core_axis_name="c", subcore_axis_name="s", num_cores=_NC)
    return pl.kernel(
        _sc_scores_body,
        mesh=mesh,
        compiler_params=pltpu.CompilerParams(
            needs_layout_passes=False, use_tc_tiling_on_sc=False),
        out_type=jax.ShapeDtypeStruct((_B * _SW,), jnp.float32),
        scratch_types=[
            pltpu.VMEM((_C,), jnp.int32),
            pltpu.VMEM((_C,), jnp.int32),
            pltpu.VMEM((_C, _E), jnp.float32),
            pltpu.VMEM((_C, _E), jnp.float32),
            pltpu.VMEM((_C * _NCTX,), jnp.int32),
            pltpu.VMEM((_C * _NCTX,), jnp.int32),
            pltpu.VMEM((_C * _NCTX, _E), jnp.float32),
            pltpu.VMEM((_C * _NCTX, _E), jnp.float32),
            pltpu.VMEM((_C * _SW,), jnp.float32),
            pltpu.SemaphoreType.DMA,
            pltpu.SemaphoreType.DMA,
            pltpu.SemaphoreType.DMA,
            pltpu.SemaphoreType.DMA,
        ],
    )


_DW = 2048
_DG = (_V + _DW - 1) // _DW
_VP = _DG * _DW


def _detile_body(a_ref, y_ref):
    a = a_ref[...]
    y_ref[...] = jnp.concatenate(
        [a[:, :_DW // 2].T, a[:, _DW // 2:].T], axis=1)


_detile = pl.pallas_call(
    _detile_body,
    grid=(_DG,),
    in_specs=[pl.BlockSpec((_E, _DW), lambda g: (0, g))],
    out_specs=pl.BlockSpec((_DW // 2, 128), lambda g: (g, 0)),
    out_shape=jax.ShapeDtypeStruct((_VP // 2, 128), jnp.float32),
)


def _to_sc_format(table):
    return _detile(table.T).reshape(_VP, _E)


def _perm_idx(v):
    return ((v >> 11) << 11) + ((v & 1023) << 1) + ((v >> 10) & 1)


def _tc_loss_body(scores_ref, out_ref):
    s = scores_ref[...]
    col = lax.broadcasted_iota(jnp.int32, s.shape, 1)
    x = jnp.where(col < _P, s, -s)
    ls = jnp.minimum(x, 0.0) - jnp.log1p(jnp.exp(-jnp.abs(x)))
    val = jnp.where(col < _NCTX, ls, 0.0)
    out_ref[...] = -jnp.sum(val, axis=1, keepdims=True)


_tc_loss = pl.pallas_call(
    _tc_loss_body,
    out_shape=jax.ShapeDtypeStruct((_B, 1), jnp.float32),
)


def kernel(center_word, target_word, negative_word, in_embed, out_embed):
    center = _perm_idx(center_word.astype(jnp.int32))
    ctx = _perm_idx(jnp.concatenate(
        [target_word.astype(jnp.int32), negative_word.astype(jnp.int32)],
        axis=1))
    ctx_flat = ctx.reshape(-1)
    in_flat = _to_sc_format(in_embed)
    out_flat = _to_sc_format(out_embed)
    scores_flat = _sc_scores_fn()(center, ctx_flat, in_flat, out_flat)
    scores = scores_flat.reshape(_B, _SW)
    loss = _tc_loss(scores)
    return loss.reshape(_B)


# speedup vs baseline: 7.0033x; 1.5105x over previous
import functools

import jax
import jax.numpy as jnp
from jax import lax
from jax.experimental import pallas as pl
from jax.experimental.pallas import tpu as pltpu
from jax.experimental.pallas import tpu_sc as plsc

_B = 16384
_V = 1000000
_E = 64
_P = 10
_N = 50
_NCTX = _P + _N
_SW = 64

_NC = 2
_NS = 16
_L = 16
_NW = _NC * _NS
_BPW = _B // _NW
_C = 16
_CHUNKS = _BPW // _C

def _sc_scores_body(center_hbm, ctx_hbm, in_embed_hbm, out_embed_hbm,
                    scores_hbm,
                    cidx0, cidx1, crows0, crows1, idx0, idx1, rows0, rows1,
                    scores_v, semi0, semi1, semg0, semg1):
    wid = lax.axis_index("s") * _NC + lax.axis_index("c")
    base = wid * _BPW
    lane = lax.iota(jnp.int32, _L)
    cidx = (cidx0, cidx1)
    crows = (crows0, crows1)
    idx = (idx0, idx1)
    rows = (rows0, rows1)
    semi = (semi0, semi1)
    semg = (semg0, semg1)

    def fire_idx(c, b):
        b0 = base + c * _C
        pltpu.async_copy(
            ctx_hbm.at[pl.ds(b0 * _NCTX, _C * _NCTX)], idx[b], semi[b])
        pltpu.async_copy(center_hbm.at[pl.ds(b0, _C)], cidx[b], semi[b])

    def wait_idx(c, b):
        b0 = base + c * _C
        pltpu.make_async_copy(
            ctx_hbm.at[pl.ds(b0 * _NCTX, _C * _NCTX)], idx[b], semi[b]).wait()
        pltpu.make_async_copy(
            center_hbm.at[pl.ds(b0, _C)], cidx[b], semi[b]).wait()

    def fire_gather(b):
        pltpu.async_copy(out_embed_hbm.at[idx[b]], rows[b], semg[b])
        pltpu.async_copy(in_embed_hbm.at[cidx[b]], crows[b], semg[b])

    def wait_gather(b):
        pltpu.make_async_copy(out_embed_hbm.at[idx[b]], rows[b], semg[b]).wait()
        pltpu.make_async_copy(in_embed_hbm.at[cidx[b]], crows[b], semg[b]).wait()

    def compute(t, b):
        rv, cv = rows[b], crows[b]

        def elem_body(i, carry2):
            r0 = i * _NCTX
            s0 = i * _SW
            ivec = jnp.zeros((_L,), jnp.int32) + i
            jvecs = []
            for g in range(4):
                jv = r0 + g * 16 + lane
                if g == 3:
                    jv = jnp.minimum(jv, r0 + _NCTX - 1)
                jvecs.append(jv)
            accs = [[jnp.zeros((_L,), jnp.float32) for _ in range(2)]
                    for _ in range(4)]
            for e0 in range(_E):
                evec = (lane + e0) & (_E - 1)
                c_l = plsc.load_gather(cv, [ivec, evec])
                for g in range(4):
                    u = plsc.load_gather(rv, [jvecs[g], evec])
                    accs[g][e0 % 2] = accs[g][e0 % 2] + u * c_l
            for g in range(4):
                scores_v[pl.ds(s0 + g * 16, 16)] = accs[g][0] + accs[g][1]
            return carry2

        lax.fori_loop(0, _C, elem_body, 0)
        b0 = base + t * _C
        pltpu.sync_copy(scores_v, scores_hbm.at[pl.ds(b0 * _SW, _C * _SW)])

    fire_idx(0, 0)
    wait_idx(0, 0)
    fire_gather(0)
    fire_idx(1, 1)

    def pair_body(k, carry):
        for b in (0, 1):
            t = 2 * k + b
            nb = 1 - b

            @pl.when(t + 1 < _CHUNKS)
            def _():
                wait_idx(t + 1, nb)
                fire_gather(nb)

            wait_gather(b)

            @pl.when(t + 2 < _CHUNKS)
            def _():
                fire_idx(t + 2, b)

            compute(t, b)
        return carry

    lax.fori_loop(0, _CHUNKS // 2, pair_body, 0)


@functools.cache
def _sc_scores_fn():
    mesh = plsc.VectorSubcoreMesh(
        core_axis_name="c", subcore_axis_name="s", num_cores=_NC)
    return pl.kernel(
        _sc_scores_body,
        mesh=mesh,
        compiler_params=pltpu.CompilerParams(
            needs_layout_passes=False, use_tc_tiling_on_sc=False),
        out_type=jax.ShapeDtypeStruct((_B * _SW,), jnp.float32),
        scratch_types=[
            pltpu.VMEM((_C,), jnp.int32),
            pltpu.VMEM((_C,), jnp.int32),
            pltpu.VMEM((_C, _E), jnp.float32),
            pltpu.VMEM((_C, _E), jnp.float32),
            pltpu.VMEM((_C * _NCTX,), jnp.int32),
            pltpu.VMEM((_C * _NCTX,), jnp.int32),
            pltpu.VMEM((_C * _NCTX, _E), jnp.float32),
            pltpu.VMEM((_C * _NCTX, _E), jnp.float32),
            pltpu.VMEM((_C * _SW,), jnp.float32),
            pltpu.SemaphoreType.DMA,
            pltpu.SemaphoreType.DMA,
            pltpu.SemaphoreType.DMA,
            pltpu.SemaphoreType.DMA,
        ],
    )


_DW = 8192
_DG = (_V + _DW - 1) // _DW
_VP = _DG * _DW
_DH = _DW // 2


def _detile_body(a_ref, y_ref):
    a = a_ref[...]
    y_ref[...] = jnp.concatenate([a[:, :_DH].T, a[:, _DH:].T], axis=1)


_detile = pl.pallas_call(
    _detile_body,
    grid=(_DG,),
    in_specs=[pl.BlockSpec((_E, _DW), lambda g: (0, g))],
    out_specs=pl.BlockSpec((_DW // 2, 128), lambda g: (g, 0)),
    out_shape=jax.ShapeDtypeStruct((_VP // 2, 128), jnp.float32),
)


def _to_sc_format(table):
    return _detile(table.T).reshape(_VP, _E)


def _perm_idx(v):
    return (v // _DW) * _DW + (v % _DH) * 2 + (v % _DW) // _DH


def _tc_loss_body(scores_ref, out_ref):
    s = scores_ref[...]
    col = lax.broadcasted_iota(jnp.int32, s.shape, 1)
    x = jnp.where(col < _P, s, -s)
    ls = jnp.minimum(x, 0.0) - jnp.log1p(jnp.exp(-jnp.abs(x)))
    val = jnp.where(col < _NCTX, ls, 0.0)
    out_ref[...] = -jnp.sum(val, axis=1, keepdims=True)


_tc_loss = pl.pallas_call(
    _tc_loss_body,
    out_shape=jax.ShapeDtypeStruct((_B, 1), jnp.float32),
)


def kernel(center_word, target_word, negative_word, in_embed, out_embed):
    center = _perm_idx(center_word.astype(jnp.int32))
    ctx = _perm_idx(jnp.concatenate(
        [target_word.astype(jnp.int32), negative_word.astype(jnp.int32)],
        axis=1))
    ctx_flat = ctx.reshape(-1)
    in_flat = _to_sc_format(in_embed)
    out_flat = _to_sc_format(out_embed)
    scores_flat = _sc_scores_fn()(center, ctx_flat, in_flat, out_flat)
    scores = scores_flat.reshape(_B, _SW)
    loss = _tc_loss(scores)
    return loss.reshape(_B)


# speedup vs baseline: 7.6923x; 1.0984x over previous
import functools

import jax
import jax.numpy as jnp
from jax import lax
from jax.experimental import pallas as pl
from jax.experimental.pallas import tpu as pltpu
from jax.experimental.pallas import tpu_sc as plsc

_B = 16384
_V = 1000000
_E = 64
_P = 10
_N = 50
_NCTX = _P + _N
_SW = 64

_NC = 2
_NS = 16
_L = 16
_NW = _NC * _NS
_BPW = _B // _NW
_C = 16
_CHUNKS = _BPW // _C

def _sc_scores_body(center_hbm, ctx_hbm, in_embed_hbm, out_embed_hbm,
                    scores_hbm,
                    cidx0, cidx1, crows0, crows1, idx0, idx1, rows0, rows1,
                    scores_v, semi0, semi1, semg0, semg1):
    wid = lax.axis_index("s") * _NC + lax.axis_index("c")
    base = wid * _BPW
    lane = lax.iota(jnp.int32, _L)
    cidx = (cidx0, cidx1)
    crows = (crows0, crows1)
    idx = (idx0, idx1)
    rows = (rows0, rows1)
    semi = (semi0, semi1)
    semg = (semg0, semg1)

    def fire_idx(c, b):
        b0 = base + c * _C
        pltpu.async_copy(
            ctx_hbm.at[pl.ds(b0 * _NCTX, _C * _NCTX)], idx[b], semi[b])
        pltpu.async_copy(center_hbm.at[pl.ds(b0, _C)], cidx[b], semi[b])

    def wait_idx(c, b):
        b0 = base + c * _C
        pltpu.make_async_copy(
            ctx_hbm.at[pl.ds(b0 * _NCTX, _C * _NCTX)], idx[b], semi[b]).wait()
        pltpu.make_async_copy(
            center_hbm.at[pl.ds(b0, _C)], cidx[b], semi[b]).wait()

    def fire_gather(b):
        pltpu.async_copy(out_embed_hbm.at[idx[b]], rows[b], semg[b])
        pltpu.async_copy(in_embed_hbm.at[cidx[b]], crows[b], semg[b])

    def wait_gather(b):
        pltpu.make_async_copy(out_embed_hbm.at[idx[b]], rows[b], semg[b]).wait()
        pltpu.make_async_copy(in_embed_hbm.at[cidx[b]], crows[b], semg[b]).wait()

    def compute(t, b):
        rv, cv = rows[b], crows[b]

        def elem_body(i, carry2):
            r0 = i * _NCTX
            s0 = i * _SW
            ivec = jnp.zeros((_L,), jnp.int32) + i
            jvecs = []
            for g in range(4):
                jv = r0 + g * 16 + lane
                if g == 3:
                    jv = jnp.minimum(jv, r0 + _NCTX - 1)
                jvecs.append(jv)
            accs = [[jnp.zeros((_L,), jnp.float32) for _ in range(2)]
                    for _ in range(4)]
            for e0 in range(_E):
                evec = (lane + e0) & (_E - 1)
                c_l = plsc.load_gather(cv, [ivec, evec])
                for g in range(4):
                    u = plsc.load_gather(rv, [jvecs[g], evec])
                    accs[g][e0 % 2] = accs[g][e0 % 2] + u * c_l
            for g in range(4):
                scores_v[pl.ds(s0 + g * 16, 16)] = accs[g][0] + accs[g][1]
            return carry2

        lax.fori_loop(0, _C, elem_body, 0)
        b0 = base + t * _C
        pltpu.sync_copy(scores_v, scores_hbm.at[pl.ds(b0 * _SW, _C * _SW)])

    fire_idx(0, 0)
    wait_idx(0, 0)
    fire_gather(0)
    fire_idx(1, 1)

    def pair_body(k, carry):
        for b in (0, 1):
            t = 2 * k + b
            nb = 1 - b

            @pl.when(t + 1 < _CHUNKS)
            def _():
                wait_idx(t + 1, nb)
                fire_gather(nb)

            wait_gather(b)

            @pl.when(t + 2 < _CHUNKS)
            def _():
                fire_idx(t + 2, b)

            compute(t, b)
        return carry

    lax.fori_loop(0, _CHUNKS // 2, pair_body, 0)


@functools.cache
def _sc_scores_fn():
    mesh = plsc.VectorSubcoreMesh(
        core_axis_name="c", subcore_axis_name="s", num_cores=_NC)
    return pl.kernel(
        _sc_scores_body,
        mesh=mesh,
        compiler_params=pltpu.CompilerParams(
            needs_layout_passes=False, use_tc_tiling_on_sc=False),
        out_type=jax.ShapeDtypeStruct((_B * _SW,), jnp.float32),
        scratch_types=[
            pltpu.VMEM((_C,), jnp.int32),
            pltpu.VMEM((_C,), jnp.int32),
            pltpu.VMEM((_C, _E), jnp.float32),
            pltpu.VMEM((_C, _E), jnp.float32),
            pltpu.VMEM((_C * _NCTX,), jnp.int32),
            pltpu.VMEM((_C * _NCTX,), jnp.int32),
            pltpu.VMEM((_C * _NCTX, _E), jnp.float32),
            pltpu.VMEM((_C * _NCTX, _E), jnp.float32),
            pltpu.VMEM((_C * _SW,), jnp.float32),
            pltpu.SemaphoreType.DMA,
            pltpu.SemaphoreType.DMA,
            pltpu.SemaphoreType.DMA,
            pltpu.SemaphoreType.DMA,
        ],
    )


_DW = 16384
_DG = (_V + _DW - 1) // _DW
_VP = _DG * _DW
_DH = _DW // 2


def _detile_body(a_ref, y_ref):
    a = a_ref[...]
    y_ref[...] = jnp.concatenate([a[:, :_DH].T, a[:, _DH:].T], axis=1)


_detile = pl.pallas_call(
    _detile_body,
    grid=(_DG,),
    in_specs=[pl.BlockSpec((_E, _DW), lambda g: (0, g))],
    out_specs=pl.BlockSpec((_DW // 2, 128), lambda g: (g, 0)),
    out_shape=jax.ShapeDtypeStruct((_VP // 2, 128), jnp.float32),
)


def _to_sc_format(table):
    return _detile(table.T).reshape(_VP, _E)


def _perm_idx(v):
    return (v // _DW) * _DW + (v % _DH) * 2 + (v % _DW) // _DH


def _tc_loss_body(scores_ref, out_ref):
    s = scores_ref[...]
    col = lax.broadcasted_iota(jnp.int32, s.shape, 1)
    x = jnp.where(col < _P, s, -s)
    ls = jnp.minimum(x, 0.0) - jnp.log1p(jnp.exp(-jnp.abs(x)))
    val = jnp.where(col < _NCTX, ls, 0.0)
    out_ref[...] = -jnp.sum(val, axis=1, keepdims=True)


_tc_loss = pl.pallas_call(
    _tc_loss_body,
    out_shape=jax.ShapeDtypeStruct((_B, 1), jnp.float32),
)


def kernel(center_word, target_word, negative_word, in_embed, out_embed):
    center = _perm_idx(center_word.astype(jnp.int32))
    ctx = _perm_idx(jnp.concatenate(
        [target_word.astype(jnp.int32), negative_word.astype(jnp.int32)],
        axis=1))
    ctx_flat = ctx.reshape(-1)
    in_flat = _to_sc_format(in_embed)
    out_flat = _to_sc_format(out_embed)
    scores_flat = _sc_scores_fn()(center, ctx_flat, in_flat, out_flat)
    scores = scores_flat.reshape(_B, _SW)
    loss = _tc_loss(scores)
    return loss.reshape(_B)


# speedup vs baseline: 8.0497x; 1.0465x over previous
import functools

import jax
import jax.numpy as jnp
from jax import lax
from jax.experimental import pallas as pl
from jax.experimental.pallas import tpu as pltpu
from jax.experimental.pallas import tpu_sc as plsc

_B = 16384
_V = 1000000
_E = 64
_P = 10
_N = 50
_NCTX = _P + _N
_SW = 64

_NC = 2
_NS = 16
_L = 16
_NW = _NC * _NS
_BPW = _B // _NW
_C = 16
_CHUNKS = _BPW // _C

def _sc_scores_body(center_hbm, ctx_hbm, in_embed_hbm, out_embed_hbm,
                    scores_hbm,
                    cidx0, cidx1, crows0, crows1, idx0, idx1, rows0, rows1,
                    scores_v, semi0, semi1, semg0, semg1):
    wid = lax.axis_index("s") * _NC + lax.axis_index("c")
    base = wid * _BPW
    lane = lax.iota(jnp.int32, _L)
    cidx = (cidx0, cidx1)
    crows = (crows0, crows1)
    idx = (idx0, idx1)
    rows = (rows0, rows1)
    semi = (semi0, semi1)
    semg = (semg0, semg1)

    def fire_idx(c, b):
        b0 = base + c * _C
        pltpu.async_copy(
            ctx_hbm.at[pl.ds(b0 * _NCTX, _C * _NCTX)], idx[b], semi[b])
        pltpu.async_copy(center_hbm.at[pl.ds(b0, _C)], cidx[b], semi[b])

    def wait_idx(c, b):
        b0 = base + c * _C
        pltpu.make_async_copy(
            ctx_hbm.at[pl.ds(b0 * _NCTX, _C * _NCTX)], idx[b], semi[b]).wait()
        pltpu.make_async_copy(
            center_hbm.at[pl.ds(b0, _C)], cidx[b], semi[b]).wait()

    def fire_gather(b):
        pltpu.async_copy(out_embed_hbm.at[idx[b]], rows[b], semg[b])
        pltpu.async_copy(in_embed_hbm.at[cidx[b]], crows[b], semg[b])

    def wait_gather(b):
        pltpu.make_async_copy(out_embed_hbm.at[idx[b]], rows[b], semg[b]).wait()
        pltpu.make_async_copy(in_embed_hbm.at[cidx[b]], crows[b], semg[b]).wait()

    def compute(t, b):
        rv, cv = rows[b], crows[b]

        def elem_body(i, carry2):
            r0 = i * _NCTX
            s0 = i * _SW
            ivec = jnp.zeros((_L,), jnp.int32) + i
            jvecs = []
            for g in range(4):
                jv = r0 + g * 16 + lane
                if g == 3:
                    jv = jnp.minimum(jv, r0 + _NCTX - 1)
                jvecs.append(jv)
            accs = [[jnp.zeros((_L,), jnp.float32) for _ in range(2)]
                    for _ in range(4)]
            for e0 in range(_E):
                evec = (lane + e0) & (_E - 1)
                c_l = plsc.load_gather(cv, [ivec, evec])
                for g in range(4):
                    u = plsc.load_gather(rv, [jvecs[g], evec])
                    accs[g][e0 % 2] = accs[g][e0 % 2] + u * c_l
            for g in range(4):
                scores_v[pl.ds(s0 + g * 16, 16)] = accs[g][0] + accs[g][1]
            return carry2

        lax.fori_loop(0, _C, elem_body, 0)
        b0 = base + t * _C
        pltpu.sync_copy(scores_v, scores_hbm.at[pl.ds(b0 * _SW, _C * _SW)])

    fire_idx(0, 0)
    wait_idx(0, 0)
    fire_gather(0)
    fire_idx(1, 1)

    def pair_body(k, carry):
        for b in (0, 1):
            t = 2 * k + b
            nb = 1 - b

            @pl.when(t + 1 < _CHUNKS)
            def _():
                wait_idx(t + 1, nb)
                fire_gather(nb)

            wait_gather(b)

            @pl.when(t + 2 < _CHUNKS)
            def _():
                fire_idx(t + 2, b)

            compute(t, b)
        return carry

    lax.fori_loop(0, _CHUNKS // 2, pair_body, 0)


@functools.cache
def _sc_scores_fn():
    mesh = plsc.VectorSubcoreMesh(
        core_axis_name="c", subcore_axis_name="s", num_cores=_NC)
    return pl.kernel(
        _sc_scores_body,
        mesh=mesh,
        compiler_params=pltpu.CompilerParams(
            needs_layout_passes=False, use_tc_tiling_on_sc=False),
        out_type=jax.ShapeDtypeStruct((_B * _SW,), jnp.float32),
        scratch_types=[
            pltpu.VMEM((_C,), jnp.int32),
            pltpu.VMEM((_C,), jnp.int32),
            pltpu.VMEM((_C, _E), jnp.float32),
            pltpu.VMEM((_C, _E), jnp.float32),
            pltpu.VMEM((_C * _NCTX,), jnp.int32),
            pltpu.VMEM((_C * _NCTX,), jnp.int32),
            pltpu.VMEM((_C * _NCTX, _E), jnp.float32),
            pltpu.VMEM((_C * _NCTX, _E), jnp.float32),
            pltpu.VMEM((_C * _SW,), jnp.float32),
            pltpu.SemaphoreType.DMA,
            pltpu.SemaphoreType.DMA,
            pltpu.SemaphoreType.DMA,
            pltpu.SemaphoreType.DMA,
        ],
    )


_DW = 32768
_DG = (_V + _DW - 1) // _DW
_VP = _DG * _DW
_DH = _DW // 2


def _detile_body(a_ref, y_ref):
    a = a_ref[...]
    y_ref[...] = jnp.concatenate([a[:, :_DH].T, a[:, _DH:].T], axis=1)


_detile = pl.pallas_call(
    _detile_body,
    grid=(_DG,),
    in_specs=[pl.BlockSpec((_E, _DW), lambda g: (0, g))],
    out_specs=pl.BlockSpec((_DW // 2, 128), lambda g: (g, 0)),
    out_shape=jax.ShapeDtypeStruct((_VP // 2, 128), jnp.float32),
)


def _to_sc_format(table):
    return _detile(table.T).reshape(_VP, _E)


def _perm_idx(v):
    return (v // _DW) * _DW + (v % _DH) * 2 + (v % _DW) // _DH


def _tc_loss_body(scores_ref, out_ref):
    s = scores_ref[...]
    col = lax.broadcasted_iota(jnp.int32, s.shape, 1)
    x = jnp.where(col < _P, s, -s)
    ls = jnp.minimum(x, 0.0) - jnp.log1p(jnp.exp(-jnp.abs(x)))
    val = jnp.where(col < _NCTX, ls, 0.0)
    out_ref[...] = -jnp.sum(val, axis=1, keepdims=True)


_tc_loss = pl.pallas_call(
    _tc_loss_body,
    out_shape=jax.ShapeDtypeStruct((_B, 1), jnp.float32),
)


def kernel(center_word, target_word, negative_word, in_embed, out_embed):
    center = _perm_idx(center_word.astype(jnp.int32))
    ctx = _perm_idx(jnp.concatenate(
        [target_word.astype(jnp.int32), negative_word.astype(jnp.int32)],
        axis=1))
    ctx_flat = ctx.reshape(-1)
    in_flat = _to_sc_format(in_embed)
    out_flat = _to_sc_format(out_embed)
    scores_flat = _sc_scores_fn()(center, ctx_flat, in_flat, out_flat)
    scores = scores_flat.reshape(_B, _SW)
    loss = _tc_loss(scores)
    return loss.reshape(_B)
